# baseline (speedup 1.0000x reference)
import functools

import jax
import jax.numpy as jnp
from jax import lax
from jax.experimental import pallas as pl
from jax.experimental.pallas import tpu as pltpu
from jax.experimental.pallas import tpu_sc as plsc

N_XCOND = 64
D = 128
BATCH = 16384

_NC = 2
_NS = 16
_NW = _NC * _NS
_BPW = BATCH // _NW
_CHUNK = 64
_NCHUNK = _BPW // _CHUNK
_L = 16


@functools.partial(
    pl.kernel,
    out_type=jax.ShapeDtypeStruct((BATCH, D), jnp.float32),
    mesh=plsc.VectorSubcoreMesh(core_axis_name="c", subcore_axis_name="s"),
    scratch_types=[
        pltpu.VMEM((_BPW,), jnp.int32),
        pltpu.VMEM((_BPW,), jnp.int32),
        pltpu.VMEM((_NCHUNK, _CHUNK), jnp.int32),
        pltpu.VMEM((_BPW, D), jnp.float32),
        pltpu.SemaphoreType.DMA,
        pltpu.SemaphoreType.DMA,
    ],
)
def _embed_gather(cls_hbm, xc_hbm, table_hbm, out_hbm, cls_v, xc_v, idx_v,
                  rows_v, gsem, ssem):
    wid = lax.axis_index("s") * _NC + lax.axis_index("c")
    base = wid * _BPW

    pltpu.sync_copy(cls_hbm.at[pl.ds(base, _BPW)], cls_v)
    pltpu.sync_copy(xc_hbm.at[pl.ds(base, _BPW)], xc_v)

    gathers = []
    for j in range(_NCHUNK):
        for i in range(_CHUNK // _L):
            sl = pl.ds(j * _CHUNK + i * _L, _L)
            c = cls_v[sl]
            x = xc_v[sl]
            idx_v[j, pl.ds(i * _L, _L)] = c * N_XCOND + x
        gathers.append(
            pltpu.async_copy(
                table_hbm.at[idx_v.at[j]],
                rows_v.at[pl.ds(j * _CHUNK, _CHUNK)],
                gsem,
            )
        )

    stores = []
    for j in range(_NCHUNK):
        gathers[j].wait()
        stores.append(
            pltpu.async_copy(
                rows_v.at[pl.ds(j * _CHUNK, _CHUNK)],
                out_hbm.at[pl.ds(base + j * _CHUNK, _CHUNK)],
                ssem,
            )
        )
    for s in stores:
        s.wait()


def kernel(class_ws, x_cond_ws, embedding_table):
    return _embed_gather(class_ws, x_cond_ws, embedding_table)


# speedup vs baseline: 1.0347x; 1.0347x over previous
import functools

import jax
import jax.numpy as jnp
from jax import lax
from jax.experimental import pallas as pl
from jax.experimental.pallas import tpu as pltpu
from jax.experimental.pallas import tpu_sc as plsc

N_XCOND = 64
D = 128
BATCH = 16384

_NC = 2
_NS = 16
_NW = _NC * _NS
_BPW = BATCH // _NW
_CHUNK = 128
_NCHUNK = _BPW // _CHUNK
_L = 16


@functools.partial(
    pl.kernel,
    out_type=jax.ShapeDtypeStruct((BATCH, D), jnp.float32),
    mesh=plsc.VectorSubcoreMesh(core_axis_name="c", subcore_axis_name="s"),
    scratch_types=[
        pltpu.VMEM((_BPW,), jnp.int32),
        pltpu.VMEM((_BPW,), jnp.int32),
        pltpu.VMEM((_NCHUNK, _CHUNK), jnp.int32),
        pltpu.VMEM((_BPW, D), jnp.float32),
        pltpu.SemaphoreType.DMA,
        pltpu.SemaphoreType.DMA,
    ],
)
def _embed_gather(cls_hbm, xc_hbm, table_hbm, out_hbm, cls_v, xc_v, idx_v,
                  rows_v, gsem, ssem):
    wid = lax.axis_index("s") * _NC + lax.axis_index("c")
    base = wid * _BPW

    lc = pltpu.async_copy(cls_hbm.at[pl.ds(base, _BPW)], cls_v, gsem)
    lx = pltpu.async_copy(xc_hbm.at[pl.ds(base, _BPW)], xc_v, gsem)
    lc.wait()
    lx.wait()

    gathers = []
    for j in range(_NCHUNK):
        for i in range(_CHUNK // _L):
            sl = pl.ds(j * _CHUNK + i * _L, _L)
            c = cls_v[sl]
            x = xc_v[sl]
            idx_v[j, pl.ds(i * _L, _L)] = c * N_XCOND + x
        gathers.append(
            pltpu.async_copy(
                table_hbm.at[idx_v.at[j]],
                rows_v.at[pl.ds(j * _CHUNK, _CHUNK)],
                gsem,
            )
        )

    stores = []
    for j in range(_NCHUNK):
        gathers[j].wait()
        stores.append(
            pltpu.async_copy(
                rows_v.at[pl.ds(j * _CHUNK, _CHUNK)],
                out_hbm.at[pl.ds(base + j * _CHUNK, _CHUNK)],
                ssem,
            )
        )
    for s in stores:
        s.wait()


def kernel(class_ws, x_cond_ws, embedding_table):
    return _embed_gather(class_ws, x_cond_ws, embedding_table)


# speedup vs baseline: 1.0592x; 1.0237x over previous
import functools

import jax
import jax.numpy as jnp
from jax import lax
from jax.experimental import pallas as pl
from jax.experimental.pallas import tpu as pltpu
from jax.experimental.pallas import tpu_sc as plsc

N_XCOND = 64
D = 128
BATCH = 16384

_NC = 2
_NS = 16
_NW = _NC * _NS
_BPW = BATCH // _NW
_CHUNK = 128
_NCHUNK = _BPW // _CHUNK
_L = 16


@functools.partial(
    pl.kernel,
    out_type=jax.ShapeDtypeStruct((BATCH, D), jnp.float32),
    mesh=plsc.VectorSubcoreMesh(core_axis_name="c", subcore_axis_name="s"),
    scratch_types=[
        pltpu.VMEM((_BPW,), jnp.int32),
        pltpu.VMEM((_BPW,), jnp.int32),
        pltpu.VMEM((_NCHUNK, _CHUNK), jnp.int32),
        pltpu.VMEM((_BPW, D), jnp.float32),
        pltpu.SemaphoreType.DMA,
        pltpu.SemaphoreType.DMA,
    ],
)
def _embed_gather(cls_hbm, xc_hbm, table_hbm, out_hbm, cls_v, xc_v, idx_v,
                  rows_v, gsem, ssem):
    wid = lax.axis_index("s") * _NC + lax.axis_index("c")
    base = wid * _BPW

    lc = pltpu.async_copy(cls_hbm.at[pl.ds(base, _BPW)], cls_v, gsem)
    lx = pltpu.async_copy(xc_hbm.at[pl.ds(base, _BPW)], xc_v, gsem)
    lc.wait()
    lx.wait()

    gathers = []
    for j in range(_NCHUNK):
        for i in range(_CHUNK // _L):
            sl = pl.ds(j * _CHUNK + i * _L, _L)
            c = cls_v[sl]
            x = xc_v[sl]
            idx_v[j, pl.ds(i * _L, _L)] = c * N_XCOND + x
        gathers.append(
            pltpu.async_copy(
                table_hbm.at[idx_v.at[j]],
                rows_v.at[pl.ds(j * _CHUNK, _CHUNK)],
                gsem,
            )
        )

    for g in gathers:
        g.wait()
    pltpu.async_copy(rows_v, out_hbm.at[pl.ds(base, _BPW)], ssem).wait()


def kernel(class_ws, x_cond_ws, embedding_table):
    return _embed_gather(class_ws, x_cond_ws, embedding_table)


# speedup vs baseline: 1.0707x; 1.0108x over previous
import functools

import jax
import jax.numpy as jnp
from jax import lax
from jax.experimental import pallas as pl
from jax.experimental.pallas import tpu as pltpu
from jax.experimental.pallas import tpu_sc as plsc

N_XCOND = 64
D = 128
BATCH = 16384

_NC = 2
_NS = 16
_NW = _NC * _NS
_BPW = BATCH // _NW
_CHUNK = 128
_NCHUNK = _BPW // _CHUNK
_L = 16


@functools.partial(
    pl.kernel,
    out_type=jax.ShapeDtypeStruct((BATCH, D), jnp.float32),
    mesh=plsc.VectorSubcoreMesh(core_axis_name="c", subcore_axis_name="s"),
    scratch_types=[
        pltpu.VMEM((_BPW,), jnp.int32),
        pltpu.VMEM((_BPW,), jnp.int32),
        pltpu.VMEM((_BPW,), jnp.int32),
        pltpu.VMEM((_BPW, D), jnp.float32),
        pltpu.SemaphoreType.DMA,
        pltpu.SemaphoreType.DMA,
    ],
)
def _embed_gather(cls_hbm, xc_hbm, table_hbm, out_hbm, cls_v, xc_v, idx_v,
                  rows_v, gsem, ssem):
    wid = lax.axis_index("s") * _NC + lax.axis_index("c")
    base = wid * _BPW

    lc = pltpu.async_copy(cls_hbm.at[pl.ds(base, _BPW)], cls_v, gsem)
    lx = pltpu.async_copy(xc_hbm.at[pl.ds(base, _BPW)], xc_v, gsem)
    lc.wait()
    lx.wait()

    for i in range(_BPW // _L):
        sl = pl.ds(i * _L, _L)
        idx_v[sl] = cls_v[sl] * N_XCOND + xc_v[sl]

    pltpu.async_copy(table_hbm.at[idx_v], rows_v, gsem).wait()
    pltpu.async_copy(rows_v, out_hbm.at[pl.ds(base, _BPW)], ssem).wait()


def kernel(class_ws, x_cond_ws, embedding_table):
    return _embed_gather(class_ws, x_cond_ws, embedding_table)
